# baseline (speedup 1.0000x reference)
import functools

import jax
import jax.numpy as jnp
from jax import lax
from jax.experimental import pallas as pl
from jax.experimental.pallas import tpu as pltpu
from jax.experimental.pallas import tpu_sc as plsc

N = 10000
E = 320000
IN_DIM = 128
HID = 64
EPS = 0.3
NQ = 4096

NC = 2
NS = 16
LANES = 16

KS = 80
SB = 5
K = KS * SB
EPT = E // NS
NCH = EPT // K
SROWS = 2000

_f32 = jnp.float32


def _mesh():
    return plsc.VectorSubcoreMesh(
        core_axis_name="c", subcore_axis_name="s", num_cores=NC, num_subcores=NS
    )


_SC_PARAMS = pltpu.CompilerParams(
    needs_layout_passes=False, use_tc_tiling_on_sc=False
)


def _leaky(x):
    return jnp.where(x >= 0, x, 0.3 * x)


@functools.partial(
    pl.kernel,
    out_type=jax.ShapeDtypeStruct((NC, 3 * N), _f32),
    mesh=_mesh(),
    scratch_types=[
        pltpu.VMEM_SHARED((3 * N,), _f32),
        pltpu.VMEM((SB, KS), jnp.int32),
        pltpu.VMEM((KS,), _f32),
    ],
    compiler_params=_SC_PARAMS,
)
def _deg_kernel(dflat_h, ones_h, zeros_h, out_h, acc_sp, idx_v, ones_v):
    c = lax.axis_index("c")
    s = lax.axis_index("s")
    ept2 = 3 * EPT // 2

    pltpu.sync_copy(ones_h, ones_v)

    @pl.when(s == 0)
    def _zero():
        pltpu.sync_copy(zeros_h, acc_sp)

    plsc.subcore_barrier()

    def chunk(i, cc):
        base = c * (3 * E // 2) + s * ept2 + i * K
        for j in range(SB):
            pltpu.sync_copy(dflat_h.at[pl.ds(base + j * KS, KS)], idx_v.at[j])
        for j in range(SB):
            pltpu.sync_copy(ones_v, acc_sp.at[idx_v.at[j]], add=True)
        return cc

    lax.fori_loop(0, ept2 // K, chunk, 0)
    plsc.subcore_barrier()

    @pl.when(s == 0)
    def _out():
        pltpu.sync_copy(acc_sp, out_h.at[c])


def _make_edge_kernel2(Wc, KSx, SBx, staged):
    Kx = KSx * SBx
    nch = EPT // Kx
    scratch = [
        pltpu.VMEM_SHARED((N, Wc), _f32),
        pltpu.VMEM((SBx, KSx), jnp.int32),
        pltpu.VMEM((SBx, KSx), jnp.int32),
        pltpu.VMEM((SBx, KSx), jnp.int32),
        pltpu.VMEM((SBx, KSx, Wc), _f32),
        pltpu.VMEM((SBx, KSx), _f32),
        pltpu.VMEM((N,), _f32),
        pltpu.VMEM((N,), _f32),
        pltpu.SemaphoreType.DMA,
    ]
    if staged:
        scratch.insert(0, pltpu.VMEM_SHARED((N, Wc), _f32))

    @functools.partial(
        pl.kernel,
        out_type=jax.ShapeDtypeStruct((NC, 3, N, Wc), _f32),
        mesh=_mesh(),
        scratch_types=scratch,
        compiler_params=_SC_PARAMS,
    )
    def edge_kernel(src_h, dst_h, xf_h, p_h, q_h, z_h, out_h, *refs):
        if staged:
            x_sp = refs[0]
            refs = refs[1:]
        acc_sp, src_v, dst_v, srco_v, rows_v, e_v, p_v, q_v, sem = refs
        c = lax.axis_index("c")
        s = lax.axis_index("s")
        rsl = pl.ds(s * SROWS, SROWS)

        @pl.when(s < N // SROWS)
        def _zero():
            pltpu.sync_copy(z_h, acc_sp.at[rsl])

        for g in range(3):
            if staged:
                @pl.when(s < N // SROWS)
                def _stage(g=g):
                    pltpu.sync_copy(
                        xf_h.at[pl.ds((g * NC + c) * N + s * SROWS, SROWS)],
                        x_sp.at[rsl])

            pltpu.sync_copy(p_h.at[pl.ds(g * N, N)], p_v)
            pltpu.sync_copy(q_h.at[pl.ds(g * N, N)], q_v)
            plsc.subcore_barrier()

            def chunk(i, cc, g=g):
                base = g * E + s * EPT + i * Kx
                for j in range(SBx):
                    pltpu.sync_copy(src_h.at[pl.ds(base + j * KSx, KSx)], src_v.at[j])
                    pltpu.sync_copy(dst_h.at[pl.ds(base + j * KSx, KSx)], dst_v.at[j])
                if staged:
                    cps = [
                        pltpu.async_copy(x_sp.at[src_v.at[j]], rows_v.at[j], sem)
                        for j in range(SBx)
                    ]
                else:
                    for j in range(SBx):
                        def obody(l, uu, j=j):
                            sl = pl.ds(l * LANES, LANES)
                            srco_v[j, sl] = src_v[j, sl] + (g * NC + c) * N
                            return uu

                        lax.fori_loop(0, KSx // LANES, obody, 0)
                    cps = [
                        pltpu.async_copy(xf_h.at[srco_v.at[j]], rows_v.at[j], sem)
                        for j in range(SBx)
                    ]
                for j in range(SBx):
                    def ebody(l, uu, j=j):
                        sl = pl.ds(l * LANES, LANES)
                        s16 = src_v[j, sl]
                        d16 = dst_v[j, sl]
                        pd = plsc.load_gather(p_v, [d16])
                        qs = plsc.load_gather(q_v, [s16])
                        t = pd + qs
                        e_v[j, sl] = 1.0 - 2.0 / (1.0 + jnp.exp(t + t))
                        return uu

                    lax.fori_loop(0, KSx // LANES, ebody, 0)
                for cp in cps:
                    cp.wait()
                for j in range(SBx):
                    def sbody(l, uu, j=j):
                        e16 = e_v[j, pl.ds(l * LANES, LANES)]
                        for t in range(LANES):
                            k = l * LANES + t
                            ek = e16[t]
                            for w in range(Wc // LANES):
                                sl = pl.ds(w * LANES, LANES)
                                rows_v[j, k, sl] = rows_v[j, k, sl] * ek
                        return uu

                    lax.fori_loop(0, KSx // LANES, sbody, 0)
                for j in range(SBx):
                    pltpu.sync_copy(rows_v.at[j], acc_sp.at[dst_v.at[j]], add=True)
                return cc

            lax.fori_loop(0, nch, chunk, 0)
            plsc.subcore_barrier()

            @pl.when(s < N // SROWS)
            def _writeout(g=g):
                pltpu.sync_copy(acc_sp.at[rsl], out_h.at[c, g, rsl])
                if g < 2:
                    pltpu.sync_copy(z_h, acc_sp.at[rsl])

    return edge_kernel


_edge_l1 = _make_edge_kernel2(HID // 2, KS, SB, staged=True)
_edge_l2 = _make_edge_kernel2(96, KS, 2, staged=False)


@functools.partial(
    pl.kernel,
    out_type=jax.ShapeDtypeStruct((NQ, 16), _f32),
    mesh=_mesh(),
    scratch_types=[
        pltpu.VMEM((NQ // (NC * NS),), jnp.int32),
        pltpu.VMEM((NQ // (NC * NS), 16), _f32),
        pltpu.SemaphoreType.DMA,
    ],
    compiler_params=_SC_PARAMS,
)
def _nq_gather(y_h, nodes_h, out_h, idx_v, rows_v, sem):
    c = lax.axis_index("c")
    s = lax.axis_index("s")
    wid = s * NC + c
    bpw = NQ // (NC * NS)
    b0 = wid * bpw
    pltpu.sync_copy(nodes_h.at[pl.ds(b0, bpw)], idx_v)
    pltpu.async_copy(y_h.at[idx_v], rows_v, sem).wait()
    pltpu.sync_copy(rows_v, out_h.at[pl.ds(b0, bpw)])


BN = 2000
GRID = N // BN


def _b(shape):
    return pl.BlockSpec(shape, lambda i: (0,) * len(shape))


def _rb(cols):
    return pl.BlockSpec((BN, cols), lambda i: (i, 0))


def _ab(Wc, c, g):
    return pl.BlockSpec((1, 1, BN, Wc), lambda i, c=c, g=g: (c, g, i, 0))


def _tc1(h, t1_wt, t1_b2, g1, b1, dv3):
    Wc = HID // 2

    def body(h_ref, w_ref, b_ref, g_ref, gb_ref, d_ref, raw_ref, pq_ref, xs_ref):
        r = jnp.dot(h_ref[...], w_ref[...], preferred_element_type=_f32) + b_ref[...]
        r = _leaky(r)
        raw_ref[...] = r
        pq_ref[...] = jnp.dot(r, g_ref[...], preferred_element_type=_f32) + gb_ref[...]
        for gi in range(3):
            xg = r * d_ref[gi]
            xs_ref[gi, 0] = xg[:, :Wc]
            xs_ref[gi, 1] = xg[:, Wc:]

    return pl.pallas_call(
        body,
        grid=(GRID,),
        in_specs=[_rb(IN_DIM), _b((IN_DIM, HID)), _b((1, HID)), _b((HID, 128)), _b((1, 128)),
                  pl.BlockSpec((3, BN, 1), lambda i: (0, i, 0))],
        out_specs=[_rb(HID), _rb(128),
                   pl.BlockSpec((3, NC, BN, Wc), lambda i: (0, 0, i, 0))],
        out_shape=[
            jax.ShapeDtypeStruct((N, HID), _f32),
            jax.ShapeDtypeStruct((N, 128), _f32),
            jax.ShapeDtypeStruct((3, NC, N, Wc), _f32),
        ],
    )(h, t1_wt, t1_b2, g1, b1, dv3)


def _tc2(raw1, agg, w1, w2, w3, g2, b2, dv3):
    Wc = HID // 2
    Wc2 = 3 * HID // 2

    def body(r1_ref, a10, a11, a20, a21, a30, a31, w1_ref, w2_ref, w3_ref,
             g_ref, gb_ref, d_ref, raw2_ref, pq_ref, xs_ref):
        r1 = r1_ref[...]
        hs = []
        for gi, ((ac0, ac1), w_ref) in enumerate(
                (((a10, a11), w1_ref), ((a20, a21), w2_ref), ((a30, a31), w3_ref))):
            a = jnp.concatenate([ac0[0, 0], ac1[0, 0]], axis=-1)
            z = EPS * r1 + a * d_ref[gi]
            hs.append(_leaky(jnp.dot(z, w_ref[...], preferred_element_type=_f32)))
        r2 = jnp.concatenate(hs, axis=1)
        raw2_ref[...] = r2
        pq_ref[...] = jnp.dot(r2, g_ref[...], preferred_element_type=_f32) + gb_ref[...]
        for gi in range(3):
            xg = r2 * d_ref[gi]
            xs_ref[gi, 0] = xg[:, :Wc2]
            xs_ref[gi, 1] = xg[:, Wc2:]

    return pl.pallas_call(
        body,
        grid=(GRID,),
        in_specs=[_rb(HID),
                  _ab(Wc, 0, 0), _ab(Wc, 1, 0),
                  _ab(Wc, 0, 1), _ab(Wc, 1, 1),
                  _ab(Wc, 0, 2), _ab(Wc, 1, 2),
                  _b((HID, HID)), _b((HID, HID)), _b((HID, HID)),
                  _b((3 * HID, 128)), _b((1, 128)),
                  pl.BlockSpec((3, BN, 1), lambda i: (0, i, 0))],
        out_specs=[_rb(3 * HID), _rb(128),
                   pl.BlockSpec((3, NC, BN, Wc2), lambda i: (0, 0, i, 0))],
        out_shape=[
            jax.ShapeDtypeStruct((N, 3 * HID), _f32),
            jax.ShapeDtypeStruct((N, 128), _f32),
            jax.ShapeDtypeStruct((3, NC, N, Wc2), _f32),
        ],
    )(raw1, agg, agg, agg, agg, agg, agg, w1, w2, w3, g2, b2, dv3)


def _tc3(raw2, agg, v1, v2, v3, raw0, raw1,
         wa, wb, wc, wd, we, wf, t2b, t3p, t3bp, dv3):
    H3 = 3 * HID
    Wc = H3 // 2

    def body(r2_ref, b10, b11, b20, b21, b30, b31, v1r, v2r, v3r, r0_ref, r1_ref,
             war, wbr, wcr, wdr, wer, wfr, t2br, t3pr, t3bpr, d_ref, y_ref):
        r2 = r2_ref[...]
        y1 = t2br[...]
        for gi, ((bc0, bc1), vr, wr) in enumerate(
                (((b10, b11), v1r, war), ((b20, b21), v2r, wbr),
                 ((b30, b31), v3r, wcr))):
            bfull = jnp.concatenate([bc0[0, 0], bc1[0, 0]], axis=-1)
            z = EPS * r2 + bfull * d_ref[gi]
            h2 = _leaky(jnp.dot(z, vr[...], preferred_element_type=_f32))
            y1 = y1 + jnp.dot(h2, wr[...], preferred_element_type=_f32)
        y1 = y1 + jnp.dot(r0_ref[...], wdr[...], preferred_element_type=_f32)
        y1 = y1 + jnp.dot(r1_ref[...], wer[...], preferred_element_type=_f32)
        y1 = y1 + jnp.dot(r2, wfr[...], preferred_element_type=_f32)
        y1 = _leaky(y1)
        y_ref[...] = jnp.dot(y1, t3pr[...], preferred_element_type=_f32) + t3bpr[...]

    return pl.pallas_call(
        body,
        grid=(GRID,),
        in_specs=[_rb(H3),
                  _ab(Wc, 0, 0), _ab(Wc, 1, 0),
                  _ab(Wc, 0, 1), _ab(Wc, 1, 1),
                  _ab(Wc, 0, 2), _ab(Wc, 1, 2),
                  _b((H3, HID)), _b((H3, HID)), _b((H3, HID)),
                  _rb(IN_DIM), _rb(HID),
                  _b((HID, HID)), _b((HID, HID)), _b((HID, HID)),
                  _b((IN_DIM, HID)), _b((HID, HID)), _b((H3, HID)),
                  _b((1, HID)), _b((HID, 16)), _b((1, 16)),
                  pl.BlockSpec((3, BN, 1), lambda i: (0, i, 0))],
        out_specs=[_rb(16)],
        out_shape=[jax.ShapeDtypeStruct((N, 16), _f32)],
    )(raw2, agg, agg, agg, agg, agg, agg, v1, v2, v3, raw0, raw1,
      wa, wb, wc, wd, we, wf, t2b, t3p, t3bp, dv3)[0]


def kernel(h, edge_index1, edge_index2, edge_index3, nodes,
           t1_w, t1_b, gate1_1_w, gate1_1_b, gate1_2_w, gate1_2_b,
           gate1_3_w, gate1_3_b, hw1_1, hw1_2, hw1_3,
           gate2_1_w, gate2_1_b, gate2_2_w, gate2_2_b, gate2_3_w, gate2_3_b,
           hw2_1, hw2_2, hw2_3, t2_w, t2_b, t3_w, t3_b):
    src1d = jnp.concatenate([edge_index1[0], edge_index2[0], edge_index3[0]])
    dst1d = jnp.concatenate([edge_index1[1], edge_index2[1], edge_index3[1]])
    dflat = jnp.concatenate(
        [edge_index1[1], edge_index2[1] + N, edge_index3[1] + 2 * N])

    deg2 = _deg_kernel(dflat, jnp.ones((KS,), _f32), jnp.zeros((3 * N,), _f32))
    dvec = lax.rsqrt(jnp.maximum(deg2[0] + deg2[1], 1.0))

    def gmat(gws, gbs, dim):
        cols = []
        bias = []
        for gw, gb in zip(gws, gbs):
            cols.append(gw[0, :dim])
            cols.append(gw[0, dim:])
            bias.append(jnp.zeros((1,), _f32))
            bias.append(gb)
        m = jnp.pad(jnp.stack(cols, axis=1), ((0, 0), (0, 128 - 6)))
        bv = jnp.pad(jnp.concatenate(bias), (0, 128 - 6)).reshape(1, 128)
        return m, bv

    g1, b1 = gmat((gate1_1_w, gate1_2_w, gate1_3_w),
                  (gate1_1_b, gate1_2_b, gate1_3_b), HID)
    g2, b2 = gmat((gate2_1_w, gate2_2_w, gate2_3_w),
                  (gate2_1_b, gate2_2_b, gate2_3_b), 3 * HID)

    dv3 = dvec.reshape(3, N, 1)
    raw1, pq1, xs1 = _tc1(h, t1_w.T, t1_b.reshape(1, HID), g1, b1, dv3)
    p1f = pq1[:, 0:5:2].T.reshape(-1)
    q1f = pq1[:, 1:6:2].T.reshape(-1)

    xf1 = xs1.reshape(3 * NC * N, HID // 2)
    z1 = jnp.zeros((SROWS, HID // 2), _f32)
    agg1 = _edge_l1(src1d, dst1d, xf1, p1f, q1f, z1)

    raw2, pq2, xs2 = _tc2(raw1, agg1, hw1_1, hw1_2, hw1_3, g2, b2, dv3)
    p2f = pq2[:, 0:5:2].T.reshape(-1)
    q2f = pq2[:, 1:6:2].T.reshape(-1)

    xf2 = xs2.reshape(3 * NC * N, 96)
    z2 = jnp.zeros((SROWS, 96), _f32)
    agg2 = _edge_l2(src1d, dst1d, xf2, p2f, q2f, z2)

    t3p = jnp.pad(t3_w.T, ((0, 0), (0, 16 - t3_w.shape[0])))
    t3bp = jnp.pad(t3_b, (0, 16 - t3_b.shape[0])).reshape(1, 16)
    y2 = _tc3(raw2, agg2, hw2_1, hw2_2, hw2_3, h, raw1,
              t2_w[:, 0:64].T, t2_w[:, 64:128].T, t2_w[:, 128:192].T,
              t2_w[:, 192:320].T, t2_w[:, 320:384].T, t2_w[:, 384:576].T,
              t2_b.reshape(1, HID), t3p, t3bp, dv3)

    out16 = _nq_gather(y2, nodes)
    return out16[:, :2]


# speedup vs baseline: 1.1057x; 1.1057x over previous
import functools

import jax
import jax.numpy as jnp
from jax import lax
from jax.experimental import pallas as pl
from jax.experimental.pallas import tpu as pltpu
from jax.experimental.pallas import tpu_sc as plsc

N = 10000
E = 320000
IN_DIM = 128
HID = 64
EPS = 0.3
NQ = 4096

NC = 2
NS = 16
LANES = 16

KS = 80
SB = 5
K = KS * SB
EPT = E // NS
NCH = EPT // K
SROWS = 2000

_f32 = jnp.float32


def _mesh():
    return plsc.VectorSubcoreMesh(
        core_axis_name="c", subcore_axis_name="s", num_cores=NC, num_subcores=NS
    )


_SC_PARAMS = pltpu.CompilerParams(
    needs_layout_passes=False, use_tc_tiling_on_sc=False
)


def _leaky(x):
    return jnp.where(x >= 0, x, 0.3 * x)


@functools.partial(
    pl.kernel,
    out_type=jax.ShapeDtypeStruct((NC, 3 * N), _f32),
    mesh=_mesh(),
    scratch_types=[
        pltpu.VMEM_SHARED((3 * N,), _f32),
        pltpu.VMEM((SB, KS), jnp.int32),
        pltpu.VMEM((KS,), _f32),
    ],
    compiler_params=_SC_PARAMS,
)
def _deg_kernel(dflat_h, ones_h, zeros_h, out_h, acc_sp, idx_v, ones_v):
    c = lax.axis_index("c")
    s = lax.axis_index("s")
    ept2 = 3 * EPT // 2

    pltpu.sync_copy(ones_h, ones_v)

    @pl.when(s == 0)
    def _zero():
        pltpu.sync_copy(zeros_h, acc_sp)

    plsc.subcore_barrier()

    def chunk(i, cc):
        base = c * (3 * E // 2) + s * ept2 + i * K
        for j in range(SB):
            pltpu.sync_copy(dflat_h.at[pl.ds(base + j * KS, KS)], idx_v.at[j])
        for j in range(SB):
            pltpu.sync_copy(ones_v, acc_sp.at[idx_v.at[j]], add=True)
        return cc

    lax.fori_loop(0, ept2 // K, chunk, 0)
    plsc.subcore_barrier()

    @pl.when(s == 0)
    def _out():
        pltpu.sync_copy(acc_sp, out_h.at[c])


def _make_edge_kernel2(Wc, KSx, SBx, staged, esplit=False):
    Kx = KSx * SBx
    eptx = (E // NC if esplit else E) // NS
    nch = eptx // Kx
    scratch = [
        pltpu.VMEM_SHARED((N, Wc), _f32),
        pltpu.VMEM((SBx, KSx), jnp.int32),
        pltpu.VMEM((SBx, KSx), jnp.int32),
        pltpu.VMEM((SBx, KSx), jnp.int32),
        pltpu.VMEM((SBx, KSx, Wc), _f32),
        pltpu.VMEM((SBx, KSx), _f32),
        pltpu.VMEM((N,), _f32),
        pltpu.VMEM((N,), _f32),
        pltpu.SemaphoreType.DMA,
    ]
    if staged:
        scratch.insert(0, pltpu.VMEM_SHARED((N, Wc), _f32))

    @functools.partial(
        pl.kernel,
        out_type=jax.ShapeDtypeStruct((NC, 3, N, Wc), _f32),
        mesh=_mesh(),
        scratch_types=scratch,
        compiler_params=_SC_PARAMS,
    )
    def edge_kernel(src_h, dst_h, xf_h, p_h, q_h, z_h, out_h, *refs):
        if staged:
            x_sp = refs[0]
            refs = refs[1:]
        acc_sp, src_v, dst_v, srco_v, rows_v, e_v, p_v, q_v, sem = refs
        c = lax.axis_index("c")
        s = lax.axis_index("s")
        rsl = pl.ds(s * SROWS, SROWS)

        @pl.when(s < N // SROWS)
        def _zero():
            pltpu.sync_copy(z_h, acc_sp.at[rsl])

        for g in range(3):
            if staged:
                @pl.when(s < N // SROWS)
                def _stage(g=g):
                    if esplit:
                        xoff = g * N + s * SROWS
                    else:
                        xoff = (g * NC + c) * N + s * SROWS
                    pltpu.sync_copy(xf_h.at[pl.ds(xoff, SROWS)], x_sp.at[rsl])

            pltpu.sync_copy(p_h.at[pl.ds(g * N, N)], p_v)
            pltpu.sync_copy(q_h.at[pl.ds(g * N, N)], q_v)
            plsc.subcore_barrier()

            def chunk(i, cc, g=g):
                if esplit:
                    base = g * E + c * (E // NC) + s * eptx + i * Kx
                else:
                    base = g * E + s * eptx + i * Kx
                for j in range(SBx):
                    pltpu.sync_copy(src_h.at[pl.ds(base + j * KSx, KSx)], src_v.at[j])
                    pltpu.sync_copy(dst_h.at[pl.ds(base + j * KSx, KSx)], dst_v.at[j])
                if staged:
                    cps = [
                        pltpu.async_copy(x_sp.at[src_v.at[j]], rows_v.at[j], sem)
                        for j in range(SBx)
                    ]
                else:
                    for j in range(SBx):
                        def obody(l, uu, j=j):
                            sl = pl.ds(l * LANES, LANES)
                            srco_v[j, sl] = src_v[j, sl] + (g * NC + c) * N
                            return uu

                        lax.fori_loop(0, KSx // LANES, obody, 0)
                    cps = [
                        pltpu.async_copy(xf_h.at[srco_v.at[j]], rows_v.at[j], sem)
                        for j in range(SBx)
                    ]
                for j in range(SBx):
                    def ebody(l, uu, j=j):
                        sl = pl.ds(l * LANES, LANES)
                        s16 = src_v[j, sl]
                        d16 = dst_v[j, sl]
                        pd = plsc.load_gather(p_v, [d16])
                        qs = plsc.load_gather(q_v, [s16])
                        t = pd + qs
                        e_v[j, sl] = 1.0 - 2.0 / (1.0 + jnp.exp(t + t))
                        return uu

                    lax.fori_loop(0, KSx // LANES, ebody, 0)
                for cp in cps:
                    cp.wait()
                for j in range(SBx):
                    def sbody(l, uu, j=j):
                        e16 = e_v[j, pl.ds(l * LANES, LANES)]
                        for t in range(LANES):
                            k = l * LANES + t
                            ek = e16[t]
                            for w in range(Wc // LANES):
                                sl = pl.ds(w * LANES, LANES)
                                rows_v[j, k, sl] = rows_v[j, k, sl] * ek
                        return uu

                    lax.fori_loop(0, KSx // LANES, sbody, 0)
                for j in range(SBx):
                    pltpu.sync_copy(rows_v.at[j], acc_sp.at[dst_v.at[j]], add=True)
                return cc

            lax.fori_loop(0, nch, chunk, 0)
            plsc.subcore_barrier()

            @pl.when(s < N // SROWS)
            def _writeout(g=g):
                pltpu.sync_copy(acc_sp.at[rsl], out_h.at[c, g, rsl])
                if g < 2:
                    pltpu.sync_copy(z_h, acc_sp.at[rsl])

    return edge_kernel


_edge_l1 = _make_edge_kernel2(HID, KS, SB, staged=True, esplit=True)
_edge_l2 = _make_edge_kernel2(96, KS, 2, staged=False)


@functools.partial(
    pl.kernel,
    out_type=jax.ShapeDtypeStruct((NQ, 16), _f32),
    mesh=_mesh(),
    scratch_types=[
        pltpu.VMEM((NQ // (NC * NS),), jnp.int32),
        pltpu.VMEM((NQ // (NC * NS), 16), _f32),
        pltpu.SemaphoreType.DMA,
    ],
    compiler_params=_SC_PARAMS,
)
def _nq_gather(y_h, nodes_h, out_h, idx_v, rows_v, sem):
    c = lax.axis_index("c")
    s = lax.axis_index("s")
    wid = s * NC + c
    bpw = NQ // (NC * NS)
    b0 = wid * bpw
    pltpu.sync_copy(nodes_h.at[pl.ds(b0, bpw)], idx_v)
    pltpu.async_copy(y_h.at[idx_v], rows_v, sem).wait()
    pltpu.sync_copy(rows_v, out_h.at[pl.ds(b0, bpw)])


BN = 2000
GRID = N // BN


def _b(shape):
    return pl.BlockSpec(shape, lambda i: (0,) * len(shape))


def _rb(cols):
    return pl.BlockSpec((BN, cols), lambda i: (i, 0))


def _ab(Wc, c, g):
    return pl.BlockSpec((1, 1, BN, Wc), lambda i, c=c, g=g: (c, g, i, 0))


def _tc1(h, t1_wt, t1_b2, g1, b1, dv3):
    Wc = HID // 2

    def body(h_ref, w_ref, b_ref, g_ref, gb_ref, d_ref, raw_ref, pq_ref, xs_ref):
        r = jnp.dot(h_ref[...], w_ref[...], preferred_element_type=_f32) + b_ref[...]
        r = _leaky(r)
        raw_ref[...] = r
        pq_ref[...] = jnp.dot(r, g_ref[...], preferred_element_type=_f32) + gb_ref[...]
        for gi in range(3):
            xs_ref[gi] = r * d_ref[gi]

    return pl.pallas_call(
        body,
        grid=(GRID,),
        in_specs=[_rb(IN_DIM), _b((IN_DIM, HID)), _b((1, HID)), _b((HID, 128)), _b((1, 128)),
                  pl.BlockSpec((3, BN, 1), lambda i: (0, i, 0))],
        out_specs=[_rb(HID), _rb(128),
                   pl.BlockSpec((3, BN, HID), lambda i: (0, i, 0))],
        out_shape=[
            jax.ShapeDtypeStruct((N, HID), _f32),
            jax.ShapeDtypeStruct((N, 128), _f32),
            jax.ShapeDtypeStruct((3, N, HID), _f32),
        ],
    )(h, t1_wt, t1_b2, g1, b1, dv3)


def _tc2(raw1, agg, w1, w2, w3, g2, b2, dv3):
    Wc = HID
    Wc2 = 3 * HID // 2

    def body(r1_ref, a10, a11, a20, a21, a30, a31, w1_ref, w2_ref, w3_ref,
             g_ref, gb_ref, d_ref, raw2_ref, pq_ref, xs_ref):
        r1 = r1_ref[...]
        hs = []
        for gi, ((ac0, ac1), w_ref) in enumerate(
                (((a10, a11), w1_ref), ((a20, a21), w2_ref), ((a30, a31), w3_ref))):
            a = ac0[0, 0] + ac1[0, 0]
            z = EPS * r1 + a * d_ref[gi]
            hs.append(_leaky(jnp.dot(z, w_ref[...], preferred_element_type=_f32)))
        r2 = jnp.concatenate(hs, axis=1)
        raw2_ref[...] = r2
        pq_ref[...] = jnp.dot(r2, g_ref[...], preferred_element_type=_f32) + gb_ref[...]
        for gi in range(3):
            xg = r2 * d_ref[gi]
            xs_ref[gi, 0] = xg[:, :Wc2]
            xs_ref[gi, 1] = xg[:, Wc2:]

    return pl.pallas_call(
        body,
        grid=(GRID,),
        in_specs=[_rb(HID),
                  _ab(Wc, 0, 0), _ab(Wc, 1, 0),
                  _ab(Wc, 0, 1), _ab(Wc, 1, 1),
                  _ab(Wc, 0, 2), _ab(Wc, 1, 2),
                  _b((HID, HID)), _b((HID, HID)), _b((HID, HID)),
                  _b((3 * HID, 128)), _b((1, 128)),
                  pl.BlockSpec((3, BN, 1), lambda i: (0, i, 0))],
        out_specs=[_rb(3 * HID), _rb(128),
                   pl.BlockSpec((3, NC, BN, Wc2), lambda i: (0, 0, i, 0))],
        out_shape=[
            jax.ShapeDtypeStruct((N, 3 * HID), _f32),
            jax.ShapeDtypeStruct((N, 128), _f32),
            jax.ShapeDtypeStruct((3, NC, N, Wc2), _f32),
        ],
    )(raw1, agg, agg, agg, agg, agg, agg, w1, w2, w3, g2, b2, dv3)


def _tc3(raw2, agg, v1, v2, v3, raw0, raw1,
         wa, wb, wc, wd, we, wf, t2b, t3p, t3bp, dv3):
    H3 = 3 * HID
    Wc = H3 // 2

    def body(r2_ref, b10, b11, b20, b21, b30, b31, v1r, v2r, v3r, r0_ref, r1_ref,
             war, wbr, wcr, wdr, wer, wfr, t2br, t3pr, t3bpr, d_ref, y_ref):
        r2 = r2_ref[...]
        y1 = t2br[...]
        for gi, ((bc0, bc1), vr, wr) in enumerate(
                (((b10, b11), v1r, war), ((b20, b21), v2r, wbr),
                 ((b30, b31), v3r, wcr))):
            bfull = jnp.concatenate([bc0[0, 0], bc1[0, 0]], axis=-1)
            z = EPS * r2 + bfull * d_ref[gi]
            h2 = _leaky(jnp.dot(z, vr[...], preferred_element_type=_f32))
            y1 = y1 + jnp.dot(h2, wr[...], preferred_element_type=_f32)
        y1 = y1 + jnp.dot(r0_ref[...], wdr[...], preferred_element_type=_f32)
        y1 = y1 + jnp.dot(r1_ref[...], wer[...], preferred_element_type=_f32)
        y1 = y1 + jnp.dot(r2, wfr[...], preferred_element_type=_f32)
        y1 = _leaky(y1)
        y_ref[...] = jnp.dot(y1, t3pr[...], preferred_element_type=_f32) + t3bpr[...]

    return pl.pallas_call(
        body,
        grid=(GRID,),
        in_specs=[_rb(H3),
                  _ab(Wc, 0, 0), _ab(Wc, 1, 0),
                  _ab(Wc, 0, 1), _ab(Wc, 1, 1),
                  _ab(Wc, 0, 2), _ab(Wc, 1, 2),
                  _b((H3, HID)), _b((H3, HID)), _b((H3, HID)),
                  _rb(IN_DIM), _rb(HID),
                  _b((HID, HID)), _b((HID, HID)), _b((HID, HID)),
                  _b((IN_DIM, HID)), _b((HID, HID)), _b((H3, HID)),
                  _b((1, HID)), _b((HID, 16)), _b((1, 16)),
                  pl.BlockSpec((3, BN, 1), lambda i: (0, i, 0))],
        out_specs=[_rb(16)],
        out_shape=[jax.ShapeDtypeStruct((N, 16), _f32)],
    )(raw2, agg, agg, agg, agg, agg, agg, v1, v2, v3, raw0, raw1,
      wa, wb, wc, wd, we, wf, t2b, t3p, t3bp, dv3)[0]


def kernel(h, edge_index1, edge_index2, edge_index3, nodes,
           t1_w, t1_b, gate1_1_w, gate1_1_b, gate1_2_w, gate1_2_b,
           gate1_3_w, gate1_3_b, hw1_1, hw1_2, hw1_3,
           gate2_1_w, gate2_1_b, gate2_2_w, gate2_2_b, gate2_3_w, gate2_3_b,
           hw2_1, hw2_2, hw2_3, t2_w, t2_b, t3_w, t3_b):
    src1d = jnp.concatenate([edge_index1[0], edge_index2[0], edge_index3[0]])
    dst1d = jnp.concatenate([edge_index1[1], edge_index2[1], edge_index3[1]])
    dflat = jnp.concatenate(
        [edge_index1[1], edge_index2[1] + N, edge_index3[1] + 2 * N])

    deg2 = _deg_kernel(dflat, jnp.ones((KS,), _f32), jnp.zeros((3 * N,), _f32))
    dvec = lax.rsqrt(jnp.maximum(deg2[0] + deg2[1], 1.0))

    def gmat(gws, gbs, dim):
        cols = []
        bias = []
        for gw, gb in zip(gws, gbs):
            cols.append(gw[0, :dim])
            cols.append(gw[0, dim:])
            bias.append(jnp.zeros((1,), _f32))
            bias.append(gb)
        m = jnp.pad(jnp.stack(cols, axis=1), ((0, 0), (0, 128 - 6)))
        bv = jnp.pad(jnp.concatenate(bias), (0, 128 - 6)).reshape(1, 128)
        return m, bv

    g1, b1 = gmat((gate1_1_w, gate1_2_w, gate1_3_w),
                  (gate1_1_b, gate1_2_b, gate1_3_b), HID)
    g2, b2 = gmat((gate2_1_w, gate2_2_w, gate2_3_w),
                  (gate2_1_b, gate2_2_b, gate2_3_b), 3 * HID)

    dv3 = dvec.reshape(3, N, 1)
    raw1, pq1, xs1 = _tc1(h, t1_w.T, t1_b.reshape(1, HID), g1, b1, dv3)
    p1f = pq1[:, 0:5:2].T.reshape(-1)
    q1f = pq1[:, 1:6:2].T.reshape(-1)

    xf1 = xs1.reshape(3 * N, HID)
    z1 = jnp.zeros((SROWS, HID), _f32)
    agg1 = _edge_l1(src1d, dst1d, xf1, p1f, q1f, z1)

    raw2, pq2, xs2 = _tc2(raw1, agg1, hw1_1, hw1_2, hw1_3, g2, b2, dv3)
    p2f = pq2[:, 0:5:2].T.reshape(-1)
    q2f = pq2[:, 1:6:2].T.reshape(-1)

    xf2 = xs2.reshape(3 * NC * N, 96)
    z2 = jnp.zeros((SROWS, 96), _f32)
    agg2 = _edge_l2(src1d, dst1d, xf2, p2f, q2f, z2)

    t3p = jnp.pad(t3_w.T, ((0, 0), (0, 16 - t3_w.shape[0])))
    t3bp = jnp.pad(t3_b, (0, 16 - t3_b.shape[0])).reshape(1, 16)
    y2 = _tc3(raw2, agg2, hw2_1, hw2_2, hw2_3, h, raw1,
              t2_w[:, 0:64].T, t2_w[:, 64:128].T, t2_w[:, 128:192].T,
              t2_w[:, 192:320].T, t2_w[:, 320:384].T, t2_w[:, 384:576].T,
              t2_b.reshape(1, HID), t3p, t3bp, dv3)

    out16 = _nq_gather(y2, nodes)
    return out16[:, :2]


# speedup vs baseline: 1.1607x; 1.0497x over previous
import functools

import jax
import jax.numpy as jnp
from jax import lax
from jax.experimental import pallas as pl
from jax.experimental.pallas import tpu as pltpu
from jax.experimental.pallas import tpu_sc as plsc

N = 10000
E = 320000
IN_DIM = 128
HID = 64
EPS = 0.3
NQ = 4096

NC = 2
NS = 16
LANES = 16

KS = 80
SB = 5
K = KS * SB
EPT = E // NS
NCH = EPT // K
SROWS = 2000

_f32 = jnp.float32


def _mesh():
    return plsc.VectorSubcoreMesh(
        core_axis_name="c", subcore_axis_name="s", num_cores=NC, num_subcores=NS
    )


_SC_PARAMS = pltpu.CompilerParams(
    needs_layout_passes=False, use_tc_tiling_on_sc=False
)


def _leaky(x):
    return jnp.where(x >= 0, x, 0.3 * x)


@functools.partial(
    pl.kernel,
    out_type=jax.ShapeDtypeStruct((NC, 3 * N), _f32),
    mesh=_mesh(),
    scratch_types=[
        pltpu.VMEM_SHARED((3 * N,), _f32),
        pltpu.VMEM((SB, KS), jnp.int32),
        pltpu.VMEM((KS,), _f32),
    ],
    compiler_params=_SC_PARAMS,
)
def _deg_kernel(dflat_h, ones_h, zeros_h, out_h, acc_sp, idx_v, ones_v):
    c = lax.axis_index("c")
    s = lax.axis_index("s")
    ept2 = 3 * EPT // 2

    pltpu.sync_copy(ones_h, ones_v)

    @pl.when(s == 0)
    def _zero():
        pltpu.sync_copy(zeros_h, acc_sp)

    plsc.subcore_barrier()

    def chunk(i, cc):
        base = c * (3 * E // 2) + s * ept2 + i * K
        for j in range(SB):
            pltpu.sync_copy(dflat_h.at[pl.ds(base + j * KS, KS)], idx_v.at[j])
        for j in range(SB):
            pltpu.sync_copy(ones_v, acc_sp.at[idx_v.at[j]], add=True)
        return cc

    lax.fori_loop(0, ept2 // K, chunk, 0)
    plsc.subcore_barrier()

    @pl.when(s == 0)
    def _out():
        pltpu.sync_copy(acc_sp, out_h.at[c])


def _make_edge_kernel2(Wc, KSx, SBx, staged, esplit=False):
    Kx = KSx * SBx
    eptx = (E // NC if esplit else E) // NS
    nch = eptx // Kx
    scratch = [
        pltpu.VMEM_SHARED((N, Wc), _f32),
        pltpu.VMEM((SBx, KSx), jnp.int32),
        pltpu.VMEM((SBx, KSx), jnp.int32),
        pltpu.VMEM((SBx, KSx), jnp.int32),
        pltpu.VMEM((SBx, KSx, Wc), _f32),
        pltpu.VMEM((SBx, KSx), _f32),
        pltpu.VMEM((N,), _f32),
        pltpu.VMEM((N,), _f32),
        pltpu.SemaphoreType.DMA,
    ]
    if staged:
        scratch.insert(0, pltpu.VMEM_SHARED((N, Wc), _f32))

    @functools.partial(
        pl.kernel,
        out_type=jax.ShapeDtypeStruct((NC, 3, N, Wc), _f32),
        mesh=_mesh(),
        scratch_types=scratch,
        compiler_params=_SC_PARAMS,
    )
    def edge_kernel(src_h, dst_h, xf_h, p_h, q_h, z_h, out_h, *refs):
        if staged:
            x_sp = refs[0]
            refs = refs[1:]
        acc_sp, src_v, dst_v, srco_v, rows_v, e_v, p_v, q_v, sem = refs
        c = lax.axis_index("c")
        s = lax.axis_index("s")
        rsl = pl.ds(s * SROWS, SROWS)

        @pl.when(s < N // SROWS)
        def _zero():
            pltpu.sync_copy(z_h, acc_sp.at[rsl])

        for g in range(3):
            if staged:
                @pl.when(s < N // SROWS)
                def _stage(g=g):
                    if esplit:
                        xoff = g * N + s * SROWS
                    else:
                        xoff = (g * NC + c) * N + s * SROWS
                    pltpu.sync_copy(xf_h.at[pl.ds(xoff, SROWS)], x_sp.at[rsl])

            pltpu.sync_copy(p_h.at[pl.ds(g * N, N)], p_v)
            pltpu.sync_copy(q_h.at[pl.ds(g * N, N)], q_v)
            plsc.subcore_barrier()

            def chunk(i, cc, g=g):
                if esplit:
                    base = g * E + c * (E // NC) + s * eptx + i * Kx
                else:
                    base = g * E + s * eptx + i * Kx
                for j in range(SBx):
                    pltpu.sync_copy(src_h.at[pl.ds(base + j * KSx, KSx)], src_v.at[j])
                    pltpu.sync_copy(dst_h.at[pl.ds(base + j * KSx, KSx)], dst_v.at[j])
                if staged:
                    cps = [
                        pltpu.async_copy(x_sp.at[src_v.at[j]], rows_v.at[j], sem)
                        for j in range(SBx)
                    ]
                else:
                    for j in range(SBx):
                        def obody(l, uu, j=j):
                            sl = pl.ds(l * LANES, LANES)
                            srco_v[j, sl] = src_v[j, sl] + (g * NC + c) * N
                            return uu

                        lax.fori_loop(0, KSx // LANES, obody, 0)
                    cps = [
                        pltpu.async_copy(xf_h.at[srco_v.at[j]], rows_v.at[j], sem)
                        for j in range(SBx)
                    ]
                for j in range(SBx):
                    def ebody(l, uu, j=j):
                        sl = pl.ds(l * LANES, LANES)
                        s16 = src_v[j, sl]
                        d16 = dst_v[j, sl]
                        pd = plsc.load_gather(p_v, [d16])
                        qs = plsc.load_gather(q_v, [s16])
                        t = pd + qs
                        e_v[j, sl] = 1.0 - 2.0 / (1.0 + jnp.exp(t + t))
                        return uu

                    lax.fori_loop(0, KSx // LANES, ebody, 0)
                for cp in cps:
                    cp.wait()
                for j in range(SBx):
                    def sbody(l, uu, j=j):
                        e16 = e_v[j, pl.ds(l * LANES, LANES)]
                        for t in range(LANES):
                            k = l * LANES + t
                            ek = e16[t]
                            for w in range(Wc // LANES):
                                sl = pl.ds(w * LANES, LANES)
                                rows_v[j, k, sl] = rows_v[j, k, sl] * ek
                        return uu

                    lax.fori_loop(0, KSx // LANES, sbody, 0)
                for j in range(SBx):
                    pltpu.sync_copy(rows_v.at[j], acc_sp.at[dst_v.at[j]], add=True)
                return cc

            lax.fori_loop(0, nch, chunk, 0)
            plsc.subcore_barrier()

            @pl.when(s < N // SROWS)
            def _writeout(g=g):
                pltpu.sync_copy(acc_sp.at[rsl], out_h.at[c, g, rsl])
                if g < 2:
                    pltpu.sync_copy(z_h, acc_sp.at[rsl])

    return edge_kernel


_edge_l1 = _make_edge_kernel2(HID, KS, SB, staged=True, esplit=True)
_edge_l2 = _make_edge_kernel2(96, KS, 5, staged=False)


@functools.partial(
    pl.kernel,
    out_type=jax.ShapeDtypeStruct((NQ, 16), _f32),
    mesh=_mesh(),
    scratch_types=[
        pltpu.VMEM((NQ // (NC * NS),), jnp.int32),
        pltpu.VMEM((NQ // (NC * NS), 16), _f32),
        pltpu.SemaphoreType.DMA,
    ],
    compiler_params=_SC_PARAMS,
)
def _nq_gather(y_h, nodes_h, out_h, idx_v, rows_v, sem):
    c = lax.axis_index("c")
    s = lax.axis_index("s")
    wid = s * NC + c
    bpw = NQ // (NC * NS)
    b0 = wid * bpw
    pltpu.sync_copy(nodes_h.at[pl.ds(b0, bpw)], idx_v)
    pltpu.async_copy(y_h.at[idx_v], rows_v, sem).wait()
    pltpu.sync_copy(rows_v, out_h.at[pl.ds(b0, bpw)])


BN = 2000
GRID = N // BN


def _b(shape):
    return pl.BlockSpec(shape, lambda i: (0,) * len(shape))


def _rb(cols):
    return pl.BlockSpec((BN, cols), lambda i: (i, 0))


def _ab(Wc, c, g):
    return pl.BlockSpec((1, 1, BN, Wc), lambda i, c=c, g=g: (c, g, i, 0))


def _tc1(h, t1_wt, t1_b2, g1, b1, dv3):
    Wc = HID // 2

    def body(h_ref, w_ref, b_ref, g_ref, gb_ref, d_ref, raw_ref, pq_ref, xs_ref):
        r = jnp.dot(h_ref[...], w_ref[...], preferred_element_type=_f32) + b_ref[...]
        r = _leaky(r)
        raw_ref[...] = r
        pq_ref[...] = jnp.dot(r, g_ref[...], preferred_element_type=_f32) + gb_ref[...]
        for gi in range(3):
            xs_ref[gi] = r * d_ref[gi]

    return pl.pallas_call(
        body,
        grid=(GRID,),
        in_specs=[_rb(IN_DIM), _b((IN_DIM, HID)), _b((1, HID)), _b((HID, 128)), _b((1, 128)),
                  pl.BlockSpec((3, BN, 1), lambda i: (0, i, 0))],
        out_specs=[_rb(HID), _rb(128),
                   pl.BlockSpec((3, BN, HID), lambda i: (0, i, 0))],
        out_shape=[
            jax.ShapeDtypeStruct((N, HID), _f32),
            jax.ShapeDtypeStruct((N, 128), _f32),
            jax.ShapeDtypeStruct((3, N, HID), _f32),
        ],
    )(h, t1_wt, t1_b2, g1, b1, dv3)


def _tc2(raw1, agg, w1, w2, w3, g2, b2, dv3):
    Wc = HID
    Wc2 = 3 * HID // 2

    def body(r1_ref, a10, a11, a20, a21, a30, a31, w1_ref, w2_ref, w3_ref,
             g_ref, gb_ref, d_ref, raw2_ref, pq_ref, xs_ref):
        r1 = r1_ref[...]
        hs = []
        for gi, ((ac0, ac1), w_ref) in enumerate(
                (((a10, a11), w1_ref), ((a20, a21), w2_ref), ((a30, a31), w3_ref))):
            a = ac0[0, 0] + ac1[0, 0]
            z = EPS * r1 + a * d_ref[gi]
            hs.append(_leaky(jnp.dot(z, w_ref[...], preferred_element_type=_f32)))
        r2 = jnp.concatenate(hs, axis=1)
        raw2_ref[...] = r2
        pq_ref[...] = jnp.dot(r2, g_ref[...], preferred_element_type=_f32) + gb_ref[...]
        for gi in range(3):
            xg = r2 * d_ref[gi]
            xs_ref[gi, 0] = xg[:, :Wc2]
            xs_ref[gi, 1] = xg[:, Wc2:]

    return pl.pallas_call(
        body,
        grid=(GRID,),
        in_specs=[_rb(HID),
                  _ab(Wc, 0, 0), _ab(Wc, 1, 0),
                  _ab(Wc, 0, 1), _ab(Wc, 1, 1),
                  _ab(Wc, 0, 2), _ab(Wc, 1, 2),
                  _b((HID, HID)), _b((HID, HID)), _b((HID, HID)),
                  _b((3 * HID, 128)), _b((1, 128)),
                  pl.BlockSpec((3, BN, 1), lambda i: (0, i, 0))],
        out_specs=[_rb(3 * HID), _rb(128),
                   pl.BlockSpec((3, NC, BN, Wc2), lambda i: (0, 0, i, 0))],
        out_shape=[
            jax.ShapeDtypeStruct((N, 3 * HID), _f32),
            jax.ShapeDtypeStruct((N, 128), _f32),
            jax.ShapeDtypeStruct((3, NC, N, Wc2), _f32),
        ],
    )(raw1, agg, agg, agg, agg, agg, agg, w1, w2, w3, g2, b2, dv3)


def _tc3(raw2, agg, v1, v2, v3, raw0, raw1,
         wa, wb, wc, wd, we, wf, t2b, t3p, t3bp, dv3):
    H3 = 3 * HID
    Wc = H3 // 2

    def body(r2_ref, b10, b11, b20, b21, b30, b31, v1r, v2r, v3r, r0_ref, r1_ref,
             war, wbr, wcr, wdr, wer, wfr, t2br, t3pr, t3bpr, d_ref, y_ref):
        r2 = r2_ref[...]
        y1 = t2br[...]
        for gi, ((bc0, bc1), vr, wr) in enumerate(
                (((b10, b11), v1r, war), ((b20, b21), v2r, wbr),
                 ((b30, b31), v3r, wcr))):
            bfull = jnp.concatenate([bc0[0, 0], bc1[0, 0]], axis=-1)
            z = EPS * r2 + bfull * d_ref[gi]
            h2 = _leaky(jnp.dot(z, vr[...], preferred_element_type=_f32))
            y1 = y1 + jnp.dot(h2, wr[...], preferred_element_type=_f32)
        y1 = y1 + jnp.dot(r0_ref[...], wdr[...], preferred_element_type=_f32)
        y1 = y1 + jnp.dot(r1_ref[...], wer[...], preferred_element_type=_f32)
        y1 = y1 + jnp.dot(r2, wfr[...], preferred_element_type=_f32)
        y1 = _leaky(y1)
        y_ref[...] = jnp.dot(y1, t3pr[...], preferred_element_type=_f32) + t3bpr[...]

    return pl.pallas_call(
        body,
        grid=(GRID,),
        in_specs=[_rb(H3),
                  _ab(Wc, 0, 0), _ab(Wc, 1, 0),
                  _ab(Wc, 0, 1), _ab(Wc, 1, 1),
                  _ab(Wc, 0, 2), _ab(Wc, 1, 2),
                  _b((H3, HID)), _b((H3, HID)), _b((H3, HID)),
                  _rb(IN_DIM), _rb(HID),
                  _b((HID, HID)), _b((HID, HID)), _b((HID, HID)),
                  _b((IN_DIM, HID)), _b((HID, HID)), _b((H3, HID)),
                  _b((1, HID)), _b((HID, 16)), _b((1, 16)),
                  pl.BlockSpec((3, BN, 1), lambda i: (0, i, 0))],
        out_specs=[_rb(16)],
        out_shape=[jax.ShapeDtypeStruct((N, 16), _f32)],
    )(raw2, agg, agg, agg, agg, agg, agg, v1, v2, v3, raw0, raw1,
      wa, wb, wc, wd, we, wf, t2b, t3p, t3bp, dv3)[0]


def kernel(h, edge_index1, edge_index2, edge_index3, nodes,
           t1_w, t1_b, gate1_1_w, gate1_1_b, gate1_2_w, gate1_2_b,
           gate1_3_w, gate1_3_b, hw1_1, hw1_2, hw1_3,
           gate2_1_w, gate2_1_b, gate2_2_w, gate2_2_b, gate2_3_w, gate2_3_b,
           hw2_1, hw2_2, hw2_3, t2_w, t2_b, t3_w, t3_b):
    src1d = jnp.concatenate([edge_index1[0], edge_index2[0], edge_index3[0]])
    dst1d = jnp.concatenate([edge_index1[1], edge_index2[1], edge_index3[1]])
    dflat = jnp.concatenate(
        [edge_index1[1], edge_index2[1] + N, edge_index3[1] + 2 * N])

    deg2 = _deg_kernel(dflat, jnp.ones((KS,), _f32), jnp.zeros((3 * N,), _f32))
    dvec = lax.rsqrt(jnp.maximum(deg2[0] + deg2[1], 1.0))

    def gmat(gws, gbs, dim):
        cols = []
        bias = []
        for gw, gb in zip(gws, gbs):
            cols.append(gw[0, :dim])
            cols.append(gw[0, dim:])
            bias.append(jnp.zeros((1,), _f32))
            bias.append(gb)
        m = jnp.pad(jnp.stack(cols, axis=1), ((0, 0), (0, 128 - 6)))
        bv = jnp.pad(jnp.concatenate(bias), (0, 128 - 6)).reshape(1, 128)
        return m, bv

    g1, b1 = gmat((gate1_1_w, gate1_2_w, gate1_3_w),
                  (gate1_1_b, gate1_2_b, gate1_3_b), HID)
    g2, b2 = gmat((gate2_1_w, gate2_2_w, gate2_3_w),
                  (gate2_1_b, gate2_2_b, gate2_3_b), 3 * HID)

    dv3 = dvec.reshape(3, N, 1)
    raw1, pq1, xs1 = _tc1(h, t1_w.T, t1_b.reshape(1, HID), g1, b1, dv3)
    p1f = pq1[:, 0:5:2].T.reshape(-1)
    q1f = pq1[:, 1:6:2].T.reshape(-1)

    xf1 = xs1.reshape(3 * N, HID)
    z1 = jnp.zeros((SROWS, HID), _f32)
    agg1 = _edge_l1(src1d, dst1d, xf1, p1f, q1f, z1)

    raw2, pq2, xs2 = _tc2(raw1, agg1, hw1_1, hw1_2, hw1_3, g2, b2, dv3)
    p2f = pq2[:, 0:5:2].T.reshape(-1)
    q2f = pq2[:, 1:6:2].T.reshape(-1)

    xf2 = xs2.reshape(3 * NC * N, 96)
    z2 = jnp.zeros((SROWS, 96), _f32)
    agg2 = _edge_l2(src1d, dst1d, xf2, p2f, q2f, z2)

    t3p = jnp.pad(t3_w.T, ((0, 0), (0, 16 - t3_w.shape[0])))
    t3bp = jnp.pad(t3_b, (0, 16 - t3_b.shape[0])).reshape(1, 16)
    y2 = _tc3(raw2, agg2, hw2_1, hw2_2, hw2_3, h, raw1,
              t2_w[:, 0:64].T, t2_w[:, 64:128].T, t2_w[:, 128:192].T,
              t2_w[:, 192:320].T, t2_w[:, 320:384].T, t2_w[:, 384:576].T,
              t2_b.reshape(1, HID), t3p, t3bp, dv3)

    out16 = _nq_gather(y2, nodes)
    return out16[:, :2]


# speedup vs baseline: 1.1717x; 1.0094x over previous
import functools

import jax
import jax.numpy as jnp
from jax import lax
from jax.experimental import pallas as pl
from jax.experimental.pallas import tpu as pltpu
from jax.experimental.pallas import tpu_sc as plsc

N = 10000
E = 320000
IN_DIM = 128
HID = 64
EPS = 0.3
NQ = 4096

NC = 2
NS = 16
LANES = 16

KS = 80
SB = 5
K = KS * SB
EPT = E // NS
NCH = EPT // K
SROWS = 2000

_f32 = jnp.float32


def _mesh():
    return plsc.VectorSubcoreMesh(
        core_axis_name="c", subcore_axis_name="s", num_cores=NC, num_subcores=NS
    )


_SC_PARAMS = pltpu.CompilerParams(
    needs_layout_passes=False, use_tc_tiling_on_sc=False
)


def _leaky(x):
    return jnp.where(x >= 0, x, 0.3 * x)


@functools.partial(
    pl.kernel,
    out_type=jax.ShapeDtypeStruct((NC, 3 * N), _f32),
    mesh=_mesh(),
    scratch_types=[
        pltpu.VMEM_SHARED((3 * N,), _f32),
        pltpu.VMEM((SB, KS), jnp.int32),
        pltpu.VMEM((SB, KS), jnp.int32),
        pltpu.VMEM((KS,), _f32),
    ],
    compiler_params=_SC_PARAMS,
)
def _deg_kernel(e1_h, e2_h, e3_h, ones_h, zeros_h, out_h,
                acc_sp, idx_v, idxo_v, ones_v):
    c = lax.axis_index("c")
    s = lax.axis_index("s")
    eptd = E // (NC * NS)

    pltpu.sync_copy(ones_h, ones_v)

    @pl.when(s == 0)
    def _zero():
        pltpu.sync_copy(zeros_h, acc_sp)

    plsc.subcore_barrier()

    for g, eg in enumerate((e1_h, e2_h, e3_h)):
        def chunk(i, cc, g=g, eg=eg):
            base = c * (E // NC) + s * eptd + i * K
            for j in range(SB):
                pltpu.sync_copy(eg.at[1, pl.ds(base + j * KS, KS)], idx_v.at[j])
            for j in range(SB):
                def obody(l, uu, j=j, g=g):
                    sl = pl.ds(l * LANES, LANES)
                    idxo_v[j, sl] = idx_v[j, sl] + g * N
                    return uu

                lax.fori_loop(0, KS // LANES, obody, 0)
            for j in range(SB):
                pltpu.sync_copy(ones_v, acc_sp.at[idxo_v.at[j]], add=True)
            return cc

        lax.fori_loop(0, eptd // K, chunk, 0)

    plsc.subcore_barrier()

    @pl.when(s == 0)
    def _out():
        pltpu.sync_copy(acc_sp, out_h.at[c])


def _make_edge_kernel2(Wc, KSx, SBx, staged, esplit=False):
    Kx = KSx * SBx
    eptx = (E // NC if esplit else E) // NS
    nch = eptx // Kx
    scratch = [
        pltpu.VMEM_SHARED((N, Wc), _f32),
        pltpu.VMEM((SBx, KSx), jnp.int32),
        pltpu.VMEM((SBx, KSx), jnp.int32),
        pltpu.VMEM((SBx, KSx), jnp.int32),
        pltpu.VMEM((SBx, KSx, Wc), _f32),
        pltpu.VMEM((SBx, KSx), _f32),
        pltpu.VMEM((N,), _f32),
        pltpu.VMEM((N,), _f32),
        pltpu.SemaphoreType.DMA,
    ]
    if staged:
        scratch.insert(0, pltpu.VMEM_SHARED((N, Wc), _f32))

    @functools.partial(
        pl.kernel,
        out_type=jax.ShapeDtypeStruct((NC, 3, N, Wc), _f32),
        mesh=_mesh(),
        scratch_types=scratch,
        compiler_params=_SC_PARAMS,
    )
    def edge_kernel(e1_h, e2_h, e3_h, xf_h, p_h, q_h, z_h, out_h, *refs):
        if staged:
            x_sp = refs[0]
            refs = refs[1:]
        acc_sp, src_v, dst_v, srco_v, rows_v, e_v, p_v, q_v, sem = refs
        c = lax.axis_index("c")
        s = lax.axis_index("s")
        rsl = pl.ds(s * SROWS, SROWS)

        @pl.when(s < N // SROWS)
        def _zero():
            pltpu.sync_copy(z_h, acc_sp.at[rsl])

        for g in range(3):
            if staged:
                @pl.when(s < N // SROWS)
                def _stage(g=g):
                    if esplit:
                        xoff = g * N + s * SROWS
                    else:
                        xoff = (g * NC + c) * N + s * SROWS
                    pltpu.sync_copy(xf_h.at[pl.ds(xoff, SROWS)], x_sp.at[rsl])

            pltpu.sync_copy(p_h.at[pl.ds(g * N, N)], p_v)
            pltpu.sync_copy(q_h.at[pl.ds(g * N, N)], q_v)
            plsc.subcore_barrier()

            eg = (e1_h, e2_h, e3_h)[g]

            def chunk(i, cc, g=g, eg=eg):
                if esplit:
                    base = c * (E // NC) + s * eptx + i * Kx
                else:
                    base = s * eptx + i * Kx
                for j in range(SBx):
                    pltpu.sync_copy(eg.at[0, pl.ds(base + j * KSx, KSx)], src_v.at[j])
                    pltpu.sync_copy(eg.at[1, pl.ds(base + j * KSx, KSx)], dst_v.at[j])
                if staged:
                    cps = [
                        pltpu.async_copy(x_sp.at[src_v.at[j]], rows_v.at[j], sem)
                        for j in range(SBx)
                    ]
                else:
                    for j in range(SBx):
                        def obody(l, uu, j=j):
                            sl = pl.ds(l * LANES, LANES)
                            srco_v[j, sl] = src_v[j, sl] + (g * NC + c) * N
                            return uu

                        lax.fori_loop(0, KSx // LANES, obody, 0)
                    cps = [
                        pltpu.async_copy(xf_h.at[srco_v.at[j]], rows_v.at[j], sem)
                        for j in range(SBx)
                    ]
                for j in range(SBx):
                    def ebody(l, uu, j=j):
                        sl = pl.ds(l * LANES, LANES)
                        s16 = src_v[j, sl]
                        d16 = dst_v[j, sl]
                        pd = plsc.load_gather(p_v, [d16])
                        qs = plsc.load_gather(q_v, [s16])
                        t = pd + qs
                        e_v[j, sl] = 1.0 - 2.0 / (1.0 + jnp.exp(t + t))
                        return uu

                    lax.fori_loop(0, KSx // LANES, ebody, 0)
                for cp in cps:
                    cp.wait()
                for j in range(SBx):
                    def sbody(l, uu, j=j):
                        e16 = e_v[j, pl.ds(l * LANES, LANES)]
                        for t in range(LANES):
                            k = l * LANES + t
                            ek = e16[t]
                            for w in range(Wc // LANES):
                                sl = pl.ds(w * LANES, LANES)
                                rows_v[j, k, sl] = rows_v[j, k, sl] * ek
                        return uu

                    lax.fori_loop(0, KSx // LANES, sbody, 0)
                for j in range(SBx):
                    pltpu.sync_copy(rows_v.at[j], acc_sp.at[dst_v.at[j]], add=True)
                return cc

            lax.fori_loop(0, nch, chunk, 0)
            plsc.subcore_barrier()

            @pl.when(s < N // SROWS)
            def _writeout(g=g):
                pltpu.sync_copy(acc_sp.at[rsl], out_h.at[c, g, rsl])
                if g < 2:
                    pltpu.sync_copy(z_h, acc_sp.at[rsl])

    return edge_kernel


_edge_l1 = _make_edge_kernel2(HID, KS, SB, staged=True, esplit=True)
_edge_l2 = _make_edge_kernel2(96, KS, 5, staged=False)


@functools.partial(
    pl.kernel,
    out_type=jax.ShapeDtypeStruct((NQ, 16), _f32),
    mesh=_mesh(),
    scratch_types=[
        pltpu.VMEM((NQ // (NC * NS),), jnp.int32),
        pltpu.VMEM((NQ // (NC * NS), 16), _f32),
        pltpu.SemaphoreType.DMA,
    ],
    compiler_params=_SC_PARAMS,
)
def _nq_gather(y_h, nodes_h, out_h, idx_v, rows_v, sem):
    c = lax.axis_index("c")
    s = lax.axis_index("s")
    wid = s * NC + c
    bpw = NQ // (NC * NS)
    b0 = wid * bpw
    pltpu.sync_copy(nodes_h.at[pl.ds(b0, bpw)], idx_v)
    pltpu.async_copy(y_h.at[idx_v], rows_v, sem).wait()
    pltpu.sync_copy(rows_v, out_h.at[pl.ds(b0, bpw)])


BN = 2000
GRID = N // BN


def _b(shape):
    return pl.BlockSpec(shape, lambda i: (0,) * len(shape))


def _rb(cols):
    return pl.BlockSpec((BN, cols), lambda i: (i, 0))


def _ab(Wc, c, g):
    return pl.BlockSpec((1, 1, BN, Wc), lambda i, c=c, g=g: (c, g, i, 0))


def _tc1(h, t1_wt, t1_b2, g1, b1, dv3):
    Wc = HID // 2

    def body(h_ref, w_ref, b_ref, g_ref, gb_ref, d_ref, raw_ref, pq_ref, xs_ref):
        r = jnp.dot(h_ref[...], w_ref[...], preferred_element_type=_f32) + b_ref[...]
        r = _leaky(r)
        raw_ref[...] = r
        pq_ref[...] = jnp.dot(r, g_ref[...], preferred_element_type=_f32) + gb_ref[...]
        for gi in range(3):
            xs_ref[gi] = r * d_ref[gi]

    return pl.pallas_call(
        body,
        grid=(GRID,),
        in_specs=[_rb(IN_DIM), _b((IN_DIM, HID)), _b((1, HID)), _b((HID, 128)), _b((1, 128)),
                  pl.BlockSpec((3, BN, 1), lambda i: (0, i, 0))],
        out_specs=[_rb(HID), _rb(128),
                   pl.BlockSpec((3, BN, HID), lambda i: (0, i, 0))],
        out_shape=[
            jax.ShapeDtypeStruct((N, HID), _f32),
            jax.ShapeDtypeStruct((N, 128), _f32),
            jax.ShapeDtypeStruct((3, N, HID), _f32),
        ],
    )(h, t1_wt, t1_b2, g1, b1, dv3)


def _tc2(raw1, agg, w1, w2, w3, g2, b2, dv3):
    Wc = HID
    Wc2 = 3 * HID // 2

    def body(r1_ref, a10, a11, a20, a21, a30, a31, w1_ref, w2_ref, w3_ref,
             g_ref, gb_ref, d_ref, raw2_ref, pq_ref, xs_ref):
        r1 = r1_ref[...]
        hs = []
        for gi, ((ac0, ac1), w_ref) in enumerate(
                (((a10, a11), w1_ref), ((a20, a21), w2_ref), ((a30, a31), w3_ref))):
            a = ac0[0, 0] + ac1[0, 0]
            z = EPS * r1 + a * d_ref[gi]
            hs.append(_leaky(jnp.dot(z, w_ref[...], preferred_element_type=_f32)))
        r2 = jnp.concatenate(hs, axis=1)
        raw2_ref[...] = r2
        pq_ref[...] = jnp.dot(r2, g_ref[...], preferred_element_type=_f32) + gb_ref[...]
        for gi in range(3):
            xg = r2 * d_ref[gi]
            xs_ref[gi, 0] = xg[:, :Wc2]
            xs_ref[gi, 1] = xg[:, Wc2:]

    return pl.pallas_call(
        body,
        grid=(GRID,),
        in_specs=[_rb(HID),
                  _ab(Wc, 0, 0), _ab(Wc, 1, 0),
                  _ab(Wc, 0, 1), _ab(Wc, 1, 1),
                  _ab(Wc, 0, 2), _ab(Wc, 1, 2),
                  _b((HID, HID)), _b((HID, HID)), _b((HID, HID)),
                  _b((3 * HID, 128)), _b((1, 128)),
                  pl.BlockSpec((3, BN, 1), lambda i: (0, i, 0))],
        out_specs=[_rb(3 * HID), _rb(128),
                   pl.BlockSpec((3, NC, BN, Wc2), lambda i: (0, 0, i, 0))],
        out_shape=[
            jax.ShapeDtypeStruct((N, 3 * HID), _f32),
            jax.ShapeDtypeStruct((N, 128), _f32),
            jax.ShapeDtypeStruct((3, NC, N, Wc2), _f32),
        ],
    )(raw1, agg, agg, agg, agg, agg, agg, w1, w2, w3, g2, b2, dv3)


def _tc3(raw2, agg, v1, v2, v3, raw0, raw1,
         wa, wb, wc, wd, we, wf, t2b, t3p, t3bp, dv3):
    H3 = 3 * HID
    Wc = H3 // 2

    def body(r2_ref, b10, b11, b20, b21, b30, b31, v1r, v2r, v3r, r0_ref, r1_ref,
             war, wbr, wcr, wdr, wer, wfr, t2br, t3pr, t3bpr, d_ref, y_ref):
        r2 = r2_ref[...]
        y1 = t2br[...]
        for gi, ((bc0, bc1), vr, wr) in enumerate(
                (((b10, b11), v1r, war), ((b20, b21), v2r, wbr),
                 ((b30, b31), v3r, wcr))):
            bfull = jnp.concatenate([bc0[0, 0], bc1[0, 0]], axis=-1)
            z = EPS * r2 + bfull * d_ref[gi]
            h2 = _leaky(jnp.dot(z, vr[...], preferred_element_type=_f32))
            y1 = y1 + jnp.dot(h2, wr[...], preferred_element_type=_f32)
        y1 = y1 + jnp.dot(r0_ref[...], wdr[...], preferred_element_type=_f32)
        y1 = y1 + jnp.dot(r1_ref[...], wer[...], preferred_element_type=_f32)
        y1 = y1 + jnp.dot(r2, wfr[...], preferred_element_type=_f32)
        y1 = _leaky(y1)
        y_ref[...] = jnp.dot(y1, t3pr[...], preferred_element_type=_f32) + t3bpr[...]

    return pl.pallas_call(
        body,
        grid=(GRID,),
        in_specs=[_rb(H3),
                  _ab(Wc, 0, 0), _ab(Wc, 1, 0),
                  _ab(Wc, 0, 1), _ab(Wc, 1, 1),
                  _ab(Wc, 0, 2), _ab(Wc, 1, 2),
                  _b((H3, HID)), _b((H3, HID)), _b((H3, HID)),
                  _rb(IN_DIM), _rb(HID),
                  _b((HID, HID)), _b((HID, HID)), _b((HID, HID)),
                  _b((IN_DIM, HID)), _b((HID, HID)), _b((H3, HID)),
                  _b((1, HID)), _b((HID, 16)), _b((1, 16)),
                  pl.BlockSpec((3, BN, 1), lambda i: (0, i, 0))],
        out_specs=[_rb(16)],
        out_shape=[jax.ShapeDtypeStruct((N, 16), _f32)],
    )(raw2, agg, agg, agg, agg, agg, agg, v1, v2, v3, raw0, raw1,
      wa, wb, wc, wd, we, wf, t2b, t3p, t3bp, dv3)[0]


def kernel(h, edge_index1, edge_index2, edge_index3, nodes,
           t1_w, t1_b, gate1_1_w, gate1_1_b, gate1_2_w, gate1_2_b,
           gate1_3_w, gate1_3_b, hw1_1, hw1_2, hw1_3,
           gate2_1_w, gate2_1_b, gate2_2_w, gate2_2_b, gate2_3_w, gate2_3_b,
           hw2_1, hw2_2, hw2_3, t2_w, t2_b, t3_w, t3_b):
    deg2 = _deg_kernel(edge_index1, edge_index2, edge_index3,
                       jnp.ones((KS,), _f32), jnp.zeros((3 * N,), _f32))
    dvec = lax.rsqrt(jnp.maximum(deg2[0] + deg2[1], 1.0))

    def gmat(gws, gbs, dim):
        cols = []
        bias = []
        for gw, gb in zip(gws, gbs):
            cols.append(gw[0, :dim])
            cols.append(gw[0, dim:])
            bias.append(jnp.zeros((1,), _f32))
            bias.append(gb)
        m = jnp.pad(jnp.stack(cols, axis=1), ((0, 0), (0, 128 - 6)))
        bv = jnp.pad(jnp.concatenate(bias), (0, 128 - 6)).reshape(1, 128)
        return m, bv

    g1, b1 = gmat((gate1_1_w, gate1_2_w, gate1_3_w),
                  (gate1_1_b, gate1_2_b, gate1_3_b), HID)
    g2, b2 = gmat((gate2_1_w, gate2_2_w, gate2_3_w),
                  (gate2_1_b, gate2_2_b, gate2_3_b), 3 * HID)

    dv3 = dvec.reshape(3, N, 1)
    raw1, pq1, xs1 = _tc1(h, t1_w.T, t1_b.reshape(1, HID), g1, b1, dv3)
    p1f = pq1[:, 0:5:2].T.reshape(-1)
    q1f = pq1[:, 1:6:2].T.reshape(-1)

    xf1 = xs1.reshape(3 * N, HID)
    z1 = jnp.zeros((SROWS, HID), _f32)
    agg1 = _edge_l1(edge_index1, edge_index2, edge_index3, xf1, p1f, q1f, z1)

    raw2, pq2, xs2 = _tc2(raw1, agg1, hw1_1, hw1_2, hw1_3, g2, b2, dv3)
    p2f = pq2[:, 0:5:2].T.reshape(-1)
    q2f = pq2[:, 1:6:2].T.reshape(-1)

    xf2 = xs2.reshape(3 * NC * N, 96)
    z2 = jnp.zeros((SROWS, 96), _f32)
    agg2 = _edge_l2(edge_index1, edge_index2, edge_index3, xf2, p2f, q2f, z2)

    t3p = jnp.pad(t3_w.T, ((0, 0), (0, 16 - t3_w.shape[0])))
    t3bp = jnp.pad(t3_b, (0, 16 - t3_b.shape[0])).reshape(1, 16)
    y2 = _tc3(raw2, agg2, hw2_1, hw2_2, hw2_3, h, raw1,
              t2_w[:, 0:64].T, t2_w[:, 64:128].T, t2_w[:, 128:192].T,
              t2_w[:, 192:320].T, t2_w[:, 320:384].T, t2_w[:, 384:576].T,
              t2_b.reshape(1, HID), t3p, t3bp, dv3)

    out16 = _nq_gather(y2, nodes)
    return out16[:, :2]
